# baseline (speedup 1.0000x reference)
import functools

import jax
import jax.numpy as jnp
from jax import lax
from jax.experimental import pallas as pl
from jax.experimental.pallas import tpu as pltpu
from jax.experimental.pallas import tpu_sc as plsc

NC = 2
NS = 16
L = 16
NW = NC * NS

_SPLAT_DN = lax.GatherDimensionNumbers(
    offset_dims=(), collapsed_slice_dims=(0,), start_index_map=(0,))


def _splat(vec, e):
    idx = jnp.full((L,), e, jnp.int32)
    return lax.gather(vec, idx[:, None], _SPLAT_DN, slice_sizes=(1,),
                      mode=lax.GatherScatterMode.PROMISE_IN_BOUNDS)


def _project_body(ent, rel, w1, w2, w3, watt, batt,
                  he_o, te_o, re_o, ah_o, at_o, ar_o):
    he = jnp.dot(ent[...], w1[...], preferred_element_type=jnp.float32)
    te = jnp.dot(ent[...], w3[...], preferred_element_type=jnp.float32)
    re = jnp.dot(rel[...], w2[...], preferred_element_type=jnp.float32)
    he_o[...] = he
    te_o[...] = te
    re_o[...] = re
    wa = watt[...]
    ah_o[...] = jnp.dot(he, wa, preferred_element_type=jnp.float32)
    at_o[...] = jnp.dot(te, wa, preferred_element_type=jnp.float32)
    ar_o[...] = jnp.dot(re, wa, preferred_element_type=jnp.float32) + batt[0, 0]


def _project(ent, rel_p, w1, w2, w3, watt, batt):
    n = ent.shape[0]
    rp = rel_p.shape[0]
    d = ent.shape[1]
    return pl.pallas_call(
        _project_body,
        out_shape=(
            jax.ShapeDtypeStruct((n, d), jnp.float32),
            jax.ShapeDtypeStruct((n, d), jnp.float32),
            jax.ShapeDtypeStruct((rp, d), jnp.float32),
            jax.ShapeDtypeStruct((n, 1), jnp.float32),
            jax.ShapeDtypeStruct((n, 1), jnp.float32),
            jax.ShapeDtypeStruct((rp, 1), jnp.float32),
        ),
    )(ent, rel_p, w1, w2, w3, watt, batt)


def _p_sweep_body(epw, n, rp,
                  h_hbm, r_hbm, t_hbm, ah_hbm, ar_hbm, at_hbm,
                  p_out,
                  ah_v, ar_v, at_v, h_all, r_all, t_all, p_all):
    cid = lax.axis_index("c")
    sid = lax.axis_index("s")
    wid = sid * NC + cid
    base = wid * epw

    pltpu.sync_copy(ah_hbm, ah_v)
    pltpu.sync_copy(ar_hbm, ar_v)
    pltpu.sync_copy(at_hbm, at_v)
    pltpu.sync_copy(h_hbm.at[pl.ds(base, epw)], h_all)
    pltpu.sync_copy(r_hbm.at[pl.ds(base, epw)], r_all)
    pltpu.sync_copy(t_hbm.at[pl.ds(base, epw)], t_all)

    def grp(g, _):
        sl = pl.ds(g * L, L)
        x = (plsc.load_gather(ah_v, [h_all[sl]])
             + plsc.load_gather(ar_v, [r_all[sl]])
             + plsc.load_gather(at_v, [t_all[sl]]))
        x = jnp.where(x >= 0.0, x, x * jnp.float32(0.04))
        p_all[sl] = jnp.exp(x)
        return 0

    lax.fori_loop(0, epw // L, grp, 0)
    pltpu.sync_copy(p_all, p_out.at[pl.ds(base, epw)])


def _p_sweep(h_index, r_index, t_index, ah, ar_p, at):
    e = h_index.shape[0]
    n = ah.shape[0]
    rp = ar_p.shape[0]
    epw = e // NW
    mesh = plsc.VectorSubcoreMesh(core_axis_name="c", subcore_axis_name="s")
    kern = functools.partial(
        pl.kernel,
        out_type=jax.ShapeDtypeStruct((e,), jnp.float32),
        mesh=mesh,
        compiler_params=pltpu.CompilerParams(needs_layout_passes=False),
        scratch_types=(
            pltpu.VMEM((n,), jnp.float32),
            pltpu.VMEM((rp,), jnp.float32),
            pltpu.VMEM((n,), jnp.float32),
            pltpu.VMEM((epw,), jnp.int32),
            pltpu.VMEM((epw,), jnp.int32),
            pltpu.VMEM((epw,), jnp.int32),
            pltpu.VMEM((epw,), jnp.float32),
        ),
    )(functools.partial(_p_sweep_body, epw, n, rp))
    return kern(h_index, r_index, t_index, ah, ar_p, at)


def _row_sweep_body(epw, ch, n, rp,
                    h_hbm, r_hbm, t_hbm, p_hbm, te_hbm, re_hbm,
                    acc_out, ss_out,
                    h0, r0, t0, p0, h1, r1, t1, p1,
                    re0, te0, te1, zrow,
                    acc_sh, ss_sh, re_sp,
                    sidx0, sidx1, sre0, ste0, ste1):
    cid = lax.axis_index("c")
    sid = lax.axis_index("s")
    wid = sid * NC + cid
    nchunk = epw // ch

    def issue_idx(c, hb, rb, tb, pb, sem):
        base = wid * epw + c * ch
        pltpu.async_copy(h_hbm.at[pl.ds(base, ch)], hb, sem)
        pltpu.async_copy(r_hbm.at[pl.ds(base, ch)], rb, sem)
        pltpu.async_copy(t_hbm.at[pl.ds(base, ch)], tb, sem)
        pltpu.async_copy(p_hbm.at[pl.ds(base, ch)], pb, sem)

    def wait_idx(c, hb, rb, tb, pb, sem):
        base = wid * epw + c * ch
        pltpu.make_async_copy(h_hbm.at[pl.ds(base, ch)], hb, sem).wait()
        pltpu.make_async_copy(r_hbm.at[pl.ds(base, ch)], rb, sem).wait()
        pltpu.make_async_copy(t_hbm.at[pl.ds(base, ch)], tb, sem).wait()
        pltpu.make_async_copy(p_hbm.at[pl.ds(base, ch)], pb, sem).wait()

    def issue_re(rb):
        pltpu.async_copy(re_sp.at[rb], re0, sre0)

    def wait_re(rb):
        pltpu.make_async_copy(re_sp.at[rb], re0, sre0).wait()

    def issue_te(tb, teb, ste):
        pltpu.async_copy(te_hbm.at[tb], teb, ste)

    def wait_te(tb, teb, ste):
        pltpu.make_async_copy(te_hbm.at[tb], teb, ste).wait()

    def scale(pb, teb):
        def grp(g, _):
            p16 = pb[pl.ds(g * L, L)]
            for e in range(L):
                pe = _splat(p16, e)
                row = g * L + e
                for j in range(8):
                    sl = pl.ds(j * L, L)
                    teb[row, sl] = (re0[row, sl] + teb[row, sl]) * pe
            return 0

        lax.fori_loop(0, ch // L, grp, 0)

    def scatter(hb, pb, reb):
        pltpu.sync_copy(reb, acc_sh.at[hb], add=True)
        pltpu.sync_copy(pb, ss_sh.at[hb], add=True)

    pltpu.sync_copy(re_hbm.at[pl.ds(sid * 32, 32)], re0.at[pl.ds(0, 32)])
    pltpu.sync_copy(re0.at[pl.ds(0, 32)], re_sp.at[pl.ds(sid * 32, 32)])

    zs = jnp.zeros((L,), jnp.float32)

    def zrow_body(i, _):
        for j in range(8):
            re0[i, pl.ds(j * L, L)] = zs
        return 0

    lax.fori_loop(0, ch, zrow_body, 0)

    def z1_body(i, _):
        zrow[pl.ds(i * L, L)] = zs
        return 0

    lax.fori_loop(0, 640 // L, z1_body, 0)

    zbase = sid * 624

    def zacc_body(k, _):
        pltpu.sync_copy(re0, acc_sh.at[pl.ds(zbase + k * ch, ch)])
        return 0

    lax.fori_loop(0, 640 // ch, zacc_body, 0)
    pltpu.sync_copy(zrow, ss_sh.at[pl.ds(zbase, 640)])

    plsc.subcore_barrier()

    issue_idx(0, h0, r0, t0, p0, sidx0)
    wait_idx(0, h0, r0, t0, p0, sidx0)
    issue_te(t0, te0, ste0)
    issue_re(r0)
    issue_idx(1, h1, r1, t1, p1, sidx1)

    def pipe_body(g, _):
        a = 2 * g
        wait_idx(a + 1, h1, r1, t1, p1, sidx1)
        issue_te(t1, te1, ste1)
        wait_te(t0, te0, ste0)
        wait_re(r0)
        scale(p0, te0)
        issue_re(r1)
        scatter(h0, p0, te0)
        issue_idx(a + 2, h0, r0, t0, p0, sidx0)
        wait_idx(a + 2, h0, r0, t0, p0, sidx0)
        issue_te(t0, te0, ste0)
        wait_te(t1, te1, ste1)
        wait_re(r1)
        scale(p1, te1)
        issue_re(r0)
        scatter(h1, p1, te1)

        @pl.when(a + 3 < nchunk)
        def _():
            issue_idx(a + 3, h1, r1, t1, p1, sidx1)

        return 0

    lax.fori_loop(0, (nchunk - 1) // 2, pipe_body, 0)

    wait_te(t0, te0, ste0)
    wait_re(r0)
    scale(p0, te0)
    scatter(h0, p0, te0)

    plsc.subcore_barrier()

    def wb_body(k, _):
        sl = pl.ds(zbase + k * ch, ch)
        pltpu.sync_copy(acc_sh.at[sl], re0)
        pltpu.sync_copy(re0, acc_out.at[cid, sl])
        return 0

    lax.fori_loop(0, 640 // ch, wb_body, 0)
    pltpu.sync_copy(ss_sh.at[pl.ds(zbase, 640)], zrow)
    pltpu.sync_copy(zrow, ss_out.at[pl.ds(cid * n + zbase, 640)])


def _row_sweep(h_index, r_index, t_index, p_edges, te, re_p):
    e = h_index.shape[0]
    n, d = te.shape
    rp = re_p.shape[0]
    epw = e // NW
    ch = 80
    mesh = plsc.VectorSubcoreMesh(core_axis_name="c", subcore_axis_name="s")
    kern = functools.partial(
        pl.kernel,
        out_type=(
            jax.ShapeDtypeStruct((NC, n, d), jnp.float32),
            jax.ShapeDtypeStruct((NC * n,), jnp.float32),
        ),
        mesh=mesh,
        compiler_params=pltpu.CompilerParams(needs_layout_passes=False),
        scratch_types=(
            pltpu.VMEM((ch,), jnp.int32),
            pltpu.VMEM((ch,), jnp.int32),
            pltpu.VMEM((ch,), jnp.int32),
            pltpu.VMEM((ch,), jnp.float32),
            pltpu.VMEM((ch,), jnp.int32),
            pltpu.VMEM((ch,), jnp.int32),
            pltpu.VMEM((ch,), jnp.int32),
            pltpu.VMEM((ch,), jnp.float32),
            pltpu.VMEM((ch, d), jnp.float32),
            pltpu.VMEM((ch, d), jnp.float32),
            pltpu.VMEM((ch, d), jnp.float32),
            pltpu.VMEM((640,), jnp.float32),
            pltpu.VMEM_SHARED((n, d), jnp.float32),
            pltpu.VMEM_SHARED((n,), jnp.float32),
            pltpu.VMEM_SHARED((rp, d), jnp.float32),
            pltpu.SemaphoreType.DMA,
            pltpu.SemaphoreType.DMA,
            pltpu.SemaphoreType.DMA,
            pltpu.SemaphoreType.DMA,
            pltpu.SemaphoreType.DMA,
        ),
    )(functools.partial(_row_sweep_body, epw, ch, n, rp))
    return kern(h_index, r_index, t_index, p_edges, te, re_p)


def _combine_body(acc, ss, he, bias, out):
    u = acc[0] + acc[1]
    s = (ss[0] + ss[1])[:, None]
    d = jnp.where(s > 0.0, s, jnp.float32(1.0))
    red = jnp.where(s > 0.0, he[...] + u / d, jnp.float32(0.0))
    out[...] = jnp.maximum(red + bias[...], 0.0)


def _combine(acc, ss, he, bias_row):
    n, d = he.shape
    return pl.pallas_call(
        _combine_body,
        out_shape=jax.ShapeDtypeStruct((n, d), jnp.float32),
    )(acc, ss, he, bias_row)


@jax.jit
def kernel(h_index, r_index, t_index, entity_embeddings, relation_embeddings,
           W_kernel, W_att, b_att, bias):
    n, d = entity_embeddings.shape
    r = relation_embeddings.shape[0]
    rp = (r + 15) // 16 * 16
    rel_p = jnp.pad(relation_embeddings, ((0, rp - r), (0, 0)))
    w1 = W_kernel[0:d, :]
    w2 = W_kernel[d:2 * d, :]
    w3 = W_kernel[2 * d:3 * d, :]
    batt = jnp.reshape(b_att, (1, 1))

    he, te, re_p, ah, at, ar = _project(entity_embeddings, rel_p,
                                        w1, w2, w3, W_att, batt)

    p_edges = _p_sweep(h_index, r_index, t_index,
                       jnp.reshape(ah, (n,)),
                       jnp.reshape(ar, (rp,)),
                       jnp.reshape(at, (n,)))

    acc, ss = _row_sweep(h_index, r_index, t_index, p_edges, te, re_p)

    return _combine(acc, jnp.reshape(ss, (NC, n)), he,
                    jnp.reshape(bias, (1, d)))


# speedup vs baseline: 1.1804x; 1.1804x over previous
import functools

import jax
import jax.numpy as jnp
from jax import lax
from jax.experimental import pallas as pl
from jax.experimental.pallas import tpu as pltpu
from jax.experimental.pallas import tpu_sc as plsc

NC = 2
NS = 16
L = 16
NW = NC * NS

_SPLAT_DN = lax.GatherDimensionNumbers(
    offset_dims=(), collapsed_slice_dims=(0,), start_index_map=(0,))


def _splat(vec, e):
    idx = jnp.full((L,), e, jnp.int32)
    return lax.gather(vec, idx[:, None], _SPLAT_DN, slice_sizes=(1,),
                      mode=lax.GatherScatterMode.PROMISE_IN_BOUNDS)


def _project_body(ent, rel, w1, w2, w3, watt, batt,
                  he_o, te_o, re_o, ah_o, at_o, ar_o):
    he = jnp.dot(ent[...], w1[...], preferred_element_type=jnp.float32)
    te = jnp.dot(ent[...], w3[...], preferred_element_type=jnp.float32)
    re = jnp.dot(rel[...], w2[...], preferred_element_type=jnp.float32)
    he_o[...] = he
    te_o[...] = te
    re_o[...] = re
    wa = watt[...]
    ah_o[...] = jnp.dot(he, wa, preferred_element_type=jnp.float32)
    at_o[...] = jnp.dot(te, wa, preferred_element_type=jnp.float32)
    ar_o[...] = jnp.dot(re, wa, preferred_element_type=jnp.float32) + batt[0, 0]


def _project(ent, rel_p, w1, w2, w3, watt, batt):
    n = ent.shape[0]
    rp = rel_p.shape[0]
    d = ent.shape[1]
    return pl.pallas_call(
        _project_body,
        out_shape=(
            jax.ShapeDtypeStruct((n, d), jnp.float32),
            jax.ShapeDtypeStruct((n, d), jnp.float32),
            jax.ShapeDtypeStruct((rp, d), jnp.float32),
            jax.ShapeDtypeStruct((n, 1), jnp.float32),
            jax.ShapeDtypeStruct((n, 1), jnp.float32),
            jax.ShapeDtypeStruct((rp, 1), jnp.float32),
        ),
    )(ent, rel_p, w1, w2, w3, watt, batt)


def _p_sweep_body(epw, n, rp,
                  h_hbm, r_hbm, t_hbm, ah_hbm, ar_hbm, at_hbm,
                  p_out,
                  ah_v, ar_v, at_v, h_all, r_all, t_all, p_all):
    cid = lax.axis_index("c")
    sid = lax.axis_index("s")
    wid = sid * NC + cid
    base = wid * epw

    pltpu.sync_copy(ah_hbm, ah_v)
    pltpu.sync_copy(ar_hbm, ar_v)
    pltpu.sync_copy(at_hbm, at_v)
    pltpu.sync_copy(h_hbm.at[pl.ds(base, epw)], h_all)
    pltpu.sync_copy(r_hbm.at[pl.ds(base, epw)], r_all)
    pltpu.sync_copy(t_hbm.at[pl.ds(base, epw)], t_all)

    def grp(g, _):
        sl = pl.ds(g * L, L)
        x = (plsc.load_gather(ah_v, [h_all[sl]])
             + plsc.load_gather(ar_v, [r_all[sl]])
             + plsc.load_gather(at_v, [t_all[sl]]))
        x = jnp.where(x >= 0.0, x, x * jnp.float32(0.04))
        p_all[sl] = jnp.exp(x)
        return 0

    lax.fori_loop(0, epw // L, grp, 0)
    pltpu.sync_copy(p_all, p_out.at[pl.ds(base, epw)])


def _p_sweep(h_index, r_index, t_index, ah, ar_p, at):
    e = h_index.shape[0]
    n = ah.shape[0]
    rp = ar_p.shape[0]
    epw = e // NW
    mesh = plsc.VectorSubcoreMesh(core_axis_name="c", subcore_axis_name="s")
    kern = functools.partial(
        pl.kernel,
        out_type=jax.ShapeDtypeStruct((e,), jnp.float32),
        mesh=mesh,
        compiler_params=pltpu.CompilerParams(needs_layout_passes=False),
        scratch_types=(
            pltpu.VMEM((n,), jnp.float32),
            pltpu.VMEM((rp,), jnp.float32),
            pltpu.VMEM((n,), jnp.float32),
            pltpu.VMEM((epw,), jnp.int32),
            pltpu.VMEM((epw,), jnp.int32),
            pltpu.VMEM((epw,), jnp.int32),
            pltpu.VMEM((epw,), jnp.float32),
        ),
    )(functools.partial(_p_sweep_body, epw, n, rp))
    return kern(h_index, r_index, t_index, ah, ar_p, at)


def _row_sweep_body(epw, ch, n, rp,
                    h_hbm, r_hbm, t_hbm, p_hbm, te_hbm, re_hbm,
                    acc_out, ss_out,
                    h0, r0, t0, p0, h1, r1, t1, p1,
                    re0, te0, re1, te1, zrow,
                    acc_sh, ss_sh,
                    sidx0, sidx1, sre0, ste0, sre1, ste1):
    cid = lax.axis_index("c")
    sid = lax.axis_index("s")
    wid = sid * NC + cid
    nchunk = epw // ch

    def issue_idx(c, hb, rb, tb, pb, sem):
        base = wid * epw + c * ch
        pltpu.async_copy(h_hbm.at[pl.ds(base, ch)], hb, sem)
        pltpu.async_copy(r_hbm.at[pl.ds(base, ch)], rb, sem)
        pltpu.async_copy(t_hbm.at[pl.ds(base, ch)], tb, sem)
        pltpu.async_copy(p_hbm.at[pl.ds(base, ch)], pb, sem)

    def wait_idx(c, hb, rb, tb, pb, sem):
        base = wid * epw + c * ch
        pltpu.make_async_copy(h_hbm.at[pl.ds(base, ch)], hb, sem).wait()
        pltpu.make_async_copy(r_hbm.at[pl.ds(base, ch)], rb, sem).wait()
        pltpu.make_async_copy(t_hbm.at[pl.ds(base, ch)], tb, sem).wait()
        pltpu.make_async_copy(p_hbm.at[pl.ds(base, ch)], pb, sem).wait()

    def issue_rows(rb, tb, reb, teb, sre, ste):
        pltpu.async_copy(re_hbm.at[rb], reb, sre)
        pltpu.async_copy(te_hbm.at[tb], teb, ste)

    def wait_rows(rb, tb, reb, teb, sre, ste):
        pltpu.make_async_copy(re_hbm.at[rb], reb, sre).wait()
        pltpu.make_async_copy(te_hbm.at[tb], teb, ste).wait()

    def scale(pb, reb, teb):
        @plsc.parallel_loop(0, ch // L, unroll=1)
        def grp(g):
            p16 = pb[pl.ds(g * L, L)]
            for e in range(L):
                pe = _splat(p16, e)
                row = g * L + e
                for j in range(8):
                    sl = pl.ds(j * L, L)
                    reb[row, sl] = (reb[row, sl] + teb[row, sl]) * pe

    def scatter(hb, pb, reb):
        pltpu.sync_copy(reb, acc_sh.at[hb], add=True)
        pltpu.sync_copy(pb, ss_sh.at[hb], add=True)

    zs = jnp.zeros((L,), jnp.float32)

    def zrow_body(i, _):
        for j in range(8):
            re0[i, pl.ds(j * L, L)] = zs
        return 0

    lax.fori_loop(0, ch, zrow_body, 0)

    def z1_body(i, _):
        zrow[pl.ds(i * L, L)] = zs
        return 0

    lax.fori_loop(0, 640 // L, z1_body, 0)

    zbase = sid * 624

    def zacc_body(k, _):
        pltpu.sync_copy(re0, acc_sh.at[pl.ds(zbase + k * ch, ch)])
        return 0

    lax.fori_loop(0, 640 // ch, zacc_body, 0)
    pltpu.sync_copy(zrow, ss_sh.at[pl.ds(zbase, 640)])

    plsc.subcore_barrier()

    issue_idx(0, h0, r0, t0, p0, sidx0)
    wait_idx(0, h0, r0, t0, p0, sidx0)
    issue_rows(r0, t0, re0, te0, sre0, ste0)
    issue_idx(1, h1, r1, t1, p1, sidx1)

    def pipe_body(g, _):
        a = 2 * g
        wait_idx(a + 1, h1, r1, t1, p1, sidx1)
        issue_rows(r1, t1, re1, te1, sre1, ste1)
        wait_rows(r0, t0, re0, te0, sre0, ste0)
        scale(p0, re0, te0)
        scatter(h0, p0, re0)
        issue_idx(a + 2, h0, r0, t0, p0, sidx0)
        wait_idx(a + 2, h0, r0, t0, p0, sidx0)
        issue_rows(r0, t0, re0, te0, sre0, ste0)
        wait_rows(r1, t1, re1, te1, sre1, ste1)
        scale(p1, re1, te1)
        scatter(h1, p1, re1)

        @pl.when(a + 3 < nchunk)
        def _():
            issue_idx(a + 3, h1, r1, t1, p1, sidx1)

        return 0

    lax.fori_loop(0, (nchunk - 1) // 2, pipe_body, 0)

    wait_rows(r0, t0, re0, te0, sre0, ste0)
    scale(p0, re0, te0)
    scatter(h0, p0, re0)

    plsc.subcore_barrier()

    def wb_body(k, _):
        sl = pl.ds(zbase + k * ch, ch)
        pltpu.sync_copy(acc_sh.at[sl], re0)
        pltpu.sync_copy(re0, acc_out.at[cid, sl])
        return 0

    lax.fori_loop(0, 640 // ch, wb_body, 0)
    pltpu.sync_copy(ss_sh.at[pl.ds(zbase, 640)], zrow)
    pltpu.sync_copy(zrow, ss_out.at[pl.ds(cid * n + zbase, 640)])


def _row_sweep(h_index, r_index, t_index, p_edges, te, re_p):
    e = h_index.shape[0]
    n, d = te.shape
    rp = re_p.shape[0]
    epw = e // NW
    ch = 80
    mesh = plsc.VectorSubcoreMesh(core_axis_name="c", subcore_axis_name="s")
    kern = functools.partial(
        pl.kernel,
        out_type=(
            jax.ShapeDtypeStruct((NC, n, d), jnp.float32),
            jax.ShapeDtypeStruct((NC * n,), jnp.float32),
        ),
        mesh=mesh,
        compiler_params=pltpu.CompilerParams(needs_layout_passes=False),
        scratch_types=(
            pltpu.VMEM((ch,), jnp.int32),
            pltpu.VMEM((ch,), jnp.int32),
            pltpu.VMEM((ch,), jnp.int32),
            pltpu.VMEM((ch,), jnp.float32),
            pltpu.VMEM((ch,), jnp.int32),
            pltpu.VMEM((ch,), jnp.int32),
            pltpu.VMEM((ch,), jnp.int32),
            pltpu.VMEM((ch,), jnp.float32),
            pltpu.VMEM((ch, d), jnp.float32),
            pltpu.VMEM((ch, d), jnp.float32),
            pltpu.VMEM((ch, d), jnp.float32),
            pltpu.VMEM((ch, d), jnp.float32),
            pltpu.VMEM((640,), jnp.float32),
            pltpu.VMEM_SHARED((n, d), jnp.float32),
            pltpu.VMEM_SHARED((n,), jnp.float32),
            pltpu.SemaphoreType.DMA,
            pltpu.SemaphoreType.DMA,
            pltpu.SemaphoreType.DMA,
            pltpu.SemaphoreType.DMA,
            pltpu.SemaphoreType.DMA,
            pltpu.SemaphoreType.DMA,
        ),
    )(functools.partial(_row_sweep_body, epw, ch, n, rp))
    return kern(h_index, r_index, t_index, p_edges, te, re_p)


def _combine_body(acc, ss, he, bias, out):
    u = acc[0] + acc[1]
    s = (ss[0] + ss[1])[:, None]
    d = jnp.where(s > 0.0, s, jnp.float32(1.0))
    red = jnp.where(s > 0.0, he[...] + u / d, jnp.float32(0.0))
    out[...] = jnp.maximum(red + bias[...], 0.0)


def _combine(acc, ss, he, bias_row):
    n, d = he.shape
    return pl.pallas_call(
        _combine_body,
        out_shape=jax.ShapeDtypeStruct((n, d), jnp.float32),
    )(acc, ss, he, bias_row)


@jax.jit
def kernel(h_index, r_index, t_index, entity_embeddings, relation_embeddings,
           W_kernel, W_att, b_att, bias):
    n, d = entity_embeddings.shape
    r = relation_embeddings.shape[0]
    rp = (r + 7) // 8 * 8
    rel_p = jnp.pad(relation_embeddings, ((0, rp - r), (0, 0)))
    w1 = W_kernel[0:d, :]
    w2 = W_kernel[d:2 * d, :]
    w3 = W_kernel[2 * d:3 * d, :]
    batt = jnp.reshape(b_att, (1, 1))

    he, te, re_p, ah, at, ar = _project(entity_embeddings, rel_p,
                                        w1, w2, w3, W_att, batt)

    p_edges = _p_sweep(h_index, r_index, t_index,
                       jnp.reshape(ah, (n,)),
                       jnp.reshape(ar, (rp,)),
                       jnp.reshape(at, (n,)))

    acc, ss = _row_sweep(h_index, r_index, t_index, p_edges, te, re_p)

    return _combine(acc, jnp.reshape(ss, (NC, n)), he,
                    jnp.reshape(bias, (1, d)))


# speedup vs baseline: 1.2037x; 1.0198x over previous
import functools

import jax
import jax.numpy as jnp
from jax import lax
from jax.experimental import pallas as pl
from jax.experimental.pallas import tpu as pltpu
from jax.experimental.pallas import tpu_sc as plsc

NC = 2
NS = 16
L = 16
NW = NC * NS

_SPLAT_DN = lax.GatherDimensionNumbers(
    offset_dims=(), collapsed_slice_dims=(0,), start_index_map=(0,))


def _splat(vec, e):
    idx = jnp.full((L,), e, jnp.int32)
    return lax.gather(vec, idx[:, None], _SPLAT_DN, slice_sizes=(1,),
                      mode=lax.GatherScatterMode.PROMISE_IN_BOUNDS)


def _project_body(ent, rel, w1, w2, w3, watt, batt,
                  he_o, te_o, re_o, ah_o, at_o, ar_o):
    he = jnp.dot(ent[...], w1[...], preferred_element_type=jnp.float32)
    te = jnp.dot(ent[...], w3[...], preferred_element_type=jnp.float32)
    re = jnp.dot(rel[...], w2[...], preferred_element_type=jnp.float32)
    he_o[...] = he
    te_o[...] = te
    re_o[...] = re
    wa = watt[...]
    ah_o[...] = jnp.dot(he, wa, preferred_element_type=jnp.float32)
    at_o[...] = jnp.dot(te, wa, preferred_element_type=jnp.float32)
    ar_o[...] = jnp.dot(re, wa, preferred_element_type=jnp.float32) + batt[0, 0]


def _project(ent, rel_p, w1, w2, w3, watt, batt):
    n = ent.shape[0]
    rp = rel_p.shape[0]
    d = ent.shape[1]
    return pl.pallas_call(
        _project_body,
        out_shape=(
            jax.ShapeDtypeStruct((n, d), jnp.float32),
            jax.ShapeDtypeStruct((n, d), jnp.float32),
            jax.ShapeDtypeStruct((rp, d), jnp.float32),
            jax.ShapeDtypeStruct((n, 1), jnp.float32),
            jax.ShapeDtypeStruct((n, 1), jnp.float32),
            jax.ShapeDtypeStruct((rp, 1), jnp.float32),
        ),
    )(ent, rel_p, w1, w2, w3, watt, batt)


def _p_sweep_body(epw, n, rp,
                  h_hbm, r_hbm, t_hbm, ah_hbm, ar_hbm, at_hbm,
                  p_out,
                  ah_v, ar_v, at_v, h_all, r_all, t_all, p_all):
    cid = lax.axis_index("c")
    sid = lax.axis_index("s")
    wid = sid * NC + cid
    base = wid * epw

    pltpu.sync_copy(ah_hbm, ah_v)
    pltpu.sync_copy(ar_hbm, ar_v)
    pltpu.sync_copy(at_hbm, at_v)
    pltpu.sync_copy(h_hbm.at[pl.ds(base, epw)], h_all)
    pltpu.sync_copy(r_hbm.at[pl.ds(base, epw)], r_all)
    pltpu.sync_copy(t_hbm.at[pl.ds(base, epw)], t_all)

    @plsc.parallel_loop(0, epw // L, unroll=4)
    def grp(g):
        sl = pl.ds(g * L, L)
        x = (plsc.load_gather(ah_v, [h_all[sl]])
             + plsc.load_gather(ar_v, [r_all[sl]])
             + plsc.load_gather(at_v, [t_all[sl]]))
        x = jnp.where(x >= 0.0, x, x * jnp.float32(0.04))
        p_all[sl] = jnp.exp(x)
    pltpu.sync_copy(p_all, p_out.at[pl.ds(base, epw)])


def _p_sweep(h_index, r_index, t_index, ah, ar_p, at):
    e = h_index.shape[0]
    n = ah.shape[0]
    rp = ar_p.shape[0]
    epw = e // NW
    mesh = plsc.VectorSubcoreMesh(core_axis_name="c", subcore_axis_name="s")
    kern = functools.partial(
        pl.kernel,
        out_type=jax.ShapeDtypeStruct((e,), jnp.float32),
        mesh=mesh,
        compiler_params=pltpu.CompilerParams(needs_layout_passes=False),
        scratch_types=(
            pltpu.VMEM((n,), jnp.float32),
            pltpu.VMEM((rp,), jnp.float32),
            pltpu.VMEM((n,), jnp.float32),
            pltpu.VMEM((epw,), jnp.int32),
            pltpu.VMEM((epw,), jnp.int32),
            pltpu.VMEM((epw,), jnp.int32),
            pltpu.VMEM((epw,), jnp.float32),
        ),
    )(functools.partial(_p_sweep_body, epw, n, rp))
    return kern(h_index, r_index, t_index, ah, ar_p, at)


def _row_sweep_body(epw, ch, n, rp,
                    h_hbm, r_hbm, t_hbm, p_hbm, te_hbm, re_hbm,
                    acc_out, ss_out,
                    h0, r0, t0, p0, h1, r1, t1, p1,
                    re0, te0, re1, te1, zrow,
                    acc_sh, ss_sh,
                    sidx0, sidx1, sre0, ste0, sre1, ste1):
    cid = lax.axis_index("c")
    sid = lax.axis_index("s")
    wid = sid * NC + cid
    nchunk = epw // ch

    def issue_idx(c, hb, rb, tb, pb, sem):
        base = wid * epw + c * ch
        pltpu.async_copy(h_hbm.at[pl.ds(base, ch)], hb, sem)
        pltpu.async_copy(r_hbm.at[pl.ds(base, ch)], rb, sem)
        pltpu.async_copy(t_hbm.at[pl.ds(base, ch)], tb, sem)
        pltpu.async_copy(p_hbm.at[pl.ds(base, ch)], pb, sem)

    def wait_idx(c, hb, rb, tb, pb, sem):
        base = wid * epw + c * ch
        pltpu.make_async_copy(h_hbm.at[pl.ds(base, ch)], hb, sem).wait()
        pltpu.make_async_copy(r_hbm.at[pl.ds(base, ch)], rb, sem).wait()
        pltpu.make_async_copy(t_hbm.at[pl.ds(base, ch)], tb, sem).wait()
        pltpu.make_async_copy(p_hbm.at[pl.ds(base, ch)], pb, sem).wait()

    def issue_rows(rb, tb, reb, teb, sre, ste):
        pltpu.async_copy(re_hbm.at[rb], reb, sre)
        pltpu.async_copy(te_hbm.at[tb], teb, ste)

    def wait_rows(rb, tb, reb, teb, sre, ste):
        pltpu.make_async_copy(re_hbm.at[rb], reb, sre).wait()
        pltpu.make_async_copy(te_hbm.at[tb], teb, ste).wait()

    def scale(pb, reb, teb):
        @plsc.parallel_loop(0, ch // L, unroll=1)
        def grp(g):
            p16 = pb[pl.ds(g * L, L)]
            for e in range(L):
                pe = _splat(p16, e)
                row = g * L + e
                for j in range(8):
                    sl = pl.ds(j * L, L)
                    reb[row, sl] = (reb[row, sl] + teb[row, sl]) * pe

    def scatter(hb, pb, reb):
        pltpu.sync_copy(reb, acc_sh.at[hb], add=True)
        pltpu.sync_copy(pb, ss_sh.at[hb], add=True)

    zs = jnp.zeros((L,), jnp.float32)

    def zrow_body(i, _):
        for j in range(8):
            re0[i, pl.ds(j * L, L)] = zs
        return 0

    lax.fori_loop(0, ch, zrow_body, 0)

    def z1_body(i, _):
        zrow[pl.ds(i * L, L)] = zs
        return 0

    lax.fori_loop(0, 640 // L, z1_body, 0)

    zbase = sid * 624

    def zacc_body(k, _):
        pltpu.sync_copy(re0, acc_sh.at[pl.ds(zbase + k * ch, ch)])
        return 0

    lax.fori_loop(0, 640 // ch, zacc_body, 0)
    pltpu.sync_copy(zrow, ss_sh.at[pl.ds(zbase, 640)])

    plsc.subcore_barrier()

    issue_idx(0, h0, r0, t0, p0, sidx0)
    wait_idx(0, h0, r0, t0, p0, sidx0)
    issue_rows(r0, t0, re0, te0, sre0, ste0)
    issue_idx(1, h1, r1, t1, p1, sidx1)

    def pipe_body(g, _):
        a = 2 * g
        wait_idx(a + 1, h1, r1, t1, p1, sidx1)
        issue_rows(r1, t1, re1, te1, sre1, ste1)
        wait_rows(r0, t0, re0, te0, sre0, ste0)
        scale(p0, re0, te0)
        scatter(h0, p0, re0)
        issue_idx(a + 2, h0, r0, t0, p0, sidx0)
        wait_idx(a + 2, h0, r0, t0, p0, sidx0)
        issue_rows(r0, t0, re0, te0, sre0, ste0)
        wait_rows(r1, t1, re1, te1, sre1, ste1)
        scale(p1, re1, te1)
        scatter(h1, p1, re1)

        @pl.when(a + 3 < nchunk)
        def _():
            issue_idx(a + 3, h1, r1, t1, p1, sidx1)

        return 0

    lax.fori_loop(0, (nchunk - 1) // 2, pipe_body, 0)

    wait_rows(r0, t0, re0, te0, sre0, ste0)
    scale(p0, re0, te0)
    scatter(h0, p0, re0)

    plsc.subcore_barrier()

    def wb_body(k, _):
        sl = pl.ds(zbase + k * ch, ch)
        pltpu.sync_copy(acc_sh.at[sl], re0)
        pltpu.sync_copy(re0, acc_out.at[cid, sl])
        return 0

    lax.fori_loop(0, 640 // ch, wb_body, 0)
    pltpu.sync_copy(ss_sh.at[pl.ds(zbase, 640)], zrow)
    pltpu.sync_copy(zrow, ss_out.at[pl.ds(cid * n + zbase, 640)])


def _row_sweep(h_index, r_index, t_index, p_edges, te, re_p):
    e = h_index.shape[0]
    n, d = te.shape
    rp = re_p.shape[0]
    epw = e // NW
    ch = 80
    mesh = plsc.VectorSubcoreMesh(core_axis_name="c", subcore_axis_name="s")
    kern = functools.partial(
        pl.kernel,
        out_type=(
            jax.ShapeDtypeStruct((NC, n, d), jnp.float32),
            jax.ShapeDtypeStruct((NC * n,), jnp.float32),
        ),
        mesh=mesh,
        compiler_params=pltpu.CompilerParams(needs_layout_passes=False),
        scratch_types=(
            pltpu.VMEM((ch,), jnp.int32),
            pltpu.VMEM((ch,), jnp.int32),
            pltpu.VMEM((ch,), jnp.int32),
            pltpu.VMEM((ch,), jnp.float32),
            pltpu.VMEM((ch,), jnp.int32),
            pltpu.VMEM((ch,), jnp.int32),
            pltpu.VMEM((ch,), jnp.int32),
            pltpu.VMEM((ch,), jnp.float32),
            pltpu.VMEM((ch, d), jnp.float32),
            pltpu.VMEM((ch, d), jnp.float32),
            pltpu.VMEM((ch, d), jnp.float32),
            pltpu.VMEM((ch, d), jnp.float32),
            pltpu.VMEM((640,), jnp.float32),
            pltpu.VMEM_SHARED((n, d), jnp.float32),
            pltpu.VMEM_SHARED((n,), jnp.float32),
            pltpu.SemaphoreType.DMA,
            pltpu.SemaphoreType.DMA,
            pltpu.SemaphoreType.DMA,
            pltpu.SemaphoreType.DMA,
            pltpu.SemaphoreType.DMA,
            pltpu.SemaphoreType.DMA,
        ),
    )(functools.partial(_row_sweep_body, epw, ch, n, rp))
    return kern(h_index, r_index, t_index, p_edges, te, re_p)


def _combine_body(acc, ss, he, bias, out):
    u = acc[0] + acc[1]
    s = (ss[0] + ss[1])[:, None]
    d = jnp.where(s > 0.0, s, jnp.float32(1.0))
    red = jnp.where(s > 0.0, he[...] + u / d, jnp.float32(0.0))
    out[...] = jnp.maximum(red + bias[...], 0.0)


def _combine(acc, ss, he, bias_row):
    n, d = he.shape
    return pl.pallas_call(
        _combine_body,
        out_shape=jax.ShapeDtypeStruct((n, d), jnp.float32),
    )(acc, ss, he, bias_row)


@jax.jit
def kernel(h_index, r_index, t_index, entity_embeddings, relation_embeddings,
           W_kernel, W_att, b_att, bias):
    n, d = entity_embeddings.shape
    r = relation_embeddings.shape[0]
    rp = (r + 7) // 8 * 8
    rel_p = jnp.pad(relation_embeddings, ((0, rp - r), (0, 0)))
    w1 = W_kernel[0:d, :]
    w2 = W_kernel[d:2 * d, :]
    w3 = W_kernel[2 * d:3 * d, :]
    batt = jnp.reshape(b_att, (1, 1))

    he, te, re_p, ah, at, ar = _project(entity_embeddings, rel_p,
                                        w1, w2, w3, W_att, batt)

    p_edges = _p_sweep(h_index, r_index, t_index,
                       jnp.reshape(ah, (n,)),
                       jnp.reshape(ar, (rp,)),
                       jnp.reshape(at, (n,)))

    acc, ss = _row_sweep(h_index, r_index, t_index, p_edges, te, re_p)

    return _combine(acc, jnp.reshape(ss, (NC, n)), he,
                    jnp.reshape(bias, (1, d)))
